# baseline (speedup 1.0000x reference)
import functools

import jax
import jax.numpy as jnp
import numpy as np
from jax import lax
from jax.experimental import pallas as pl
from jax.experimental.pallas import tpu as pltpu
from jax.experimental.pallas import tpu_sc as plsc

MIN_DISABLED = 2
MAX_DISABLED = 8
NEIGHBOR_PROB = 0.5


def _disabled_tofs(tof_count):
    rng = np.random.default_rng(0)
    disabled_count = int(rng.integers(MIN_DISABLED, MAX_DISABLED + 1))
    initial = int(rng.integers(0, tof_count))
    disabled = [initial]
    tof_list = [int(t) for t in rng.permutation(tof_count) if int(t) != initial]
    for _ in range(disabled_count - 1):
        rv = float(rng.random())
        perm = rng.permutation(len(disabled))
        permuted = [disabled[int(j)] for j in perm]
        if rv < NEIGHBOR_PROB:
            if rv < NEIGHBOR_PROB / 2:
                for cur in permuted:
                    new_neighbor = (cur + 1) % tof_count
                    if new_neighbor not in disabled:
                        disabled.append(new_neighbor)
                        tof_list = [t for t in tof_list if t != new_neighbor]
                        break
            else:
                opposite_found = False
                for cur in permuted:
                    new_opposite = (cur + tof_count // 2) % tof_count
                    if new_opposite not in disabled:
                        disabled.append(new_opposite)
                        tof_list = [t for t in tof_list if t != new_opposite]
                        opposite_found = True
                        break
                if not opposite_found:
                    new_element = tof_list[0]
                    tof_list = [t for t in tof_list if t != new_element]
                    disabled.append(new_element)
        else:
            new_element = tof_list[0]
            tof_list = [t for t in tof_list if t != new_element]
            disabled.append(new_element)
    return tuple(sorted(int(t) for t in disabled))


ROWS, COLS = 65536, 512
NC, NS = 2, 16
NW = NC * NS
RPW = ROWS // NW
CH = 32
NBUF = 4
CHUNKS = RPW // CH
GROUPS = CHUNKS // NBUF


def _sc_body(img_hbm, out_hbm, b0, b1, b2, b3,
             si0, si1, si2, si3, so0, so1, so2, so3):
    bufs = (b0, b1, b2, b3)
    insems = (si0, si1, si2, si3)
    outsems = (so0, so1, so2, so3)
    disabled = _disabled_tofs(COLS)

    wid = lax.axis_index("s") * NC + lax.axis_index("c")
    base = wid * RPW

    def in_cp(ci, b):
        return pltpu.make_async_copy(
            img_hbm.at[pl.ds(base + ci * CH, CH), :], bufs[b], insems[b])

    def out_cp(ci, b):
        return pltpu.make_async_copy(
            bufs[b], out_hbm.at[pl.ds(base + ci * CH, CH), :], outsems[b])

    zeros = jnp.zeros((16,), jnp.float32)
    iota = lax.iota(jnp.int32, 16)

    def zero_buf(buf):
        for c in disabled:
            col = jnp.full((16,), c, jnp.int32)
            for r0 in range(0, CH, 16):
                plsc.store_scatter(buf, [r0 + iota, col], zeros)

    in_cp(0, 0).start()
    in_cp(1, 1).start()

    def group(g, carry):
        for b in range(NBUF):
            ci = g * NBUF + b
            bp2 = (b + 2) % NBUF

            @pl.when(ci >= 2)
            def _():
                out_cp(ci - 2, bp2).wait()

            @pl.when(ci + 2 < CHUNKS)
            def _():
                in_cp(ci + 2, bp2).start()

            in_cp(ci, b).wait()
            zero_buf(bufs[b])
            out_cp(ci, b).start()
        return carry

    lax.fori_loop(0, GROUPS, group, 0)
    out_cp(CHUNKS - 2, (CHUNKS - 2) % NBUF).wait()
    out_cp(CHUNKS - 1, (CHUNKS - 1) % NBUF).wait()


@jax.jit
def kernel(img):
    mesh = plsc.VectorSubcoreMesh(core_axis_name="c", subcore_axis_name="s")
    sc_k = pl.kernel(
        _sc_body,
        out_type=jax.ShapeDtypeStruct((ROWS, COLS), jnp.float32),
        mesh=mesh,
        compiler_params=pltpu.CompilerParams(use_tc_tiling_on_sc=False, needs_layout_passes=False),
        scratch_types=(
            [pltpu.VMEM((CH, COLS), jnp.float32) for _ in range(NBUF)]
            + [pltpu.SemaphoreType.DMA for _ in range(2 * NBUF)]
        ),
    )
    return sc_k(img)


# speedup vs baseline: 3.0521x; 3.0521x over previous
import functools

import jax
import jax.numpy as jnp
import numpy as np
from jax import lax
from jax.experimental import pallas as pl
from jax.experimental.pallas import tpu as pltpu
from jax.experimental.pallas import tpu_sc as plsc

MIN_DISABLED = 2
MAX_DISABLED = 8
NEIGHBOR_PROB = 0.5


def _disabled_tofs(tof_count):
    rng = np.random.default_rng(0)
    disabled_count = int(rng.integers(MIN_DISABLED, MAX_DISABLED + 1))
    initial = int(rng.integers(0, tof_count))
    disabled = [initial]
    tof_list = [int(t) for t in rng.permutation(tof_count) if int(t) != initial]
    for _ in range(disabled_count - 1):
        rv = float(rng.random())
        perm = rng.permutation(len(disabled))
        permuted = [disabled[int(j)] for j in perm]
        if rv < NEIGHBOR_PROB:
            if rv < NEIGHBOR_PROB / 2:
                for cur in permuted:
                    new_neighbor = (cur + 1) % tof_count
                    if new_neighbor not in disabled:
                        disabled.append(new_neighbor)
                        tof_list = [t for t in tof_list if t != new_neighbor]
                        break
            else:
                opposite_found = False
                for cur in permuted:
                    new_opposite = (cur + tof_count // 2) % tof_count
                    if new_opposite not in disabled:
                        disabled.append(new_opposite)
                        tof_list = [t for t in tof_list if t != new_opposite]
                        opposite_found = True
                        break
                if not opposite_found:
                    new_element = tof_list[0]
                    tof_list = [t for t in tof_list if t != new_element]
                    disabled.append(new_element)
        else:
            new_element = tof_list[0]
            tof_list = [t for t in tof_list if t != new_element]
            disabled.append(new_element)
    return tuple(sorted(int(t) for t in disabled))


ROWS, COLS = 65536, 512
NC, NS = 2, 16
NW = NC * NS
RPW = ROWS // NW
CH = 32
NBUF = 4
CHUNKS = RPW // CH
GROUPS = CHUNKS // NBUF


def _sc_body(img_hbm, out_hbm, b0, b1, b2, b3,
             si0, si1, si2, si3, so0, so1, so2, so3):
    bufs = (b0, b1, b2, b3)
    insems = (si0, si1, si2, si3)
    outsems = (so0, so1, so2, so3)
    disabled = _disabled_tofs(COLS)

    wid = lax.axis_index("s") * NC + lax.axis_index("c")
    base = wid * RPW

    def in_cp(ci, b):
        return pltpu.make_async_copy(
            img_hbm.at[pl.ds(base + ci * CH, CH), :], bufs[b], insems[b])

    def out_cp(ci, b):
        return pltpu.make_async_copy(
            bufs[b], out_hbm.at[pl.ds(base + ci * CH, CH), :], outsems[b])

    iota = lax.iota(jnp.int32, 16)
    slabs = {}
    for c in disabled:
        slabs.setdefault((c // 16) * 16, []).append(c % 16)
    slab_masks = [
        (ca, functools.reduce(lambda a, b: a | b, [iota == l for l in lanes]))
        for ca, lanes in sorted(slabs.items())
    ]

    def zero_buf(buf):
        def row_body(r, carry):
            for ca, m in slab_masks:
                buf[r, pl.ds(ca, 16)] = jnp.where(
                    m, 0.0, buf[r, pl.ds(ca, 16)])
            return carry
        lax.fori_loop(0, CH, row_body, 0)

    in_cp(0, 0).start()
    in_cp(1, 1).start()

    def group(g, carry):
        for b in range(NBUF):
            ci = g * NBUF + b
            bp2 = (b + 2) % NBUF

            @pl.when(ci >= 2)
            def _():
                out_cp(ci - 2, bp2).wait()

            @pl.when(ci + 2 < CHUNKS)
            def _():
                in_cp(ci + 2, bp2).start()

            in_cp(ci, b).wait()
            zero_buf(bufs[b])
            out_cp(ci, b).start()
        return carry

    lax.fori_loop(0, GROUPS, group, 0)
    out_cp(CHUNKS - 2, (CHUNKS - 2) % NBUF).wait()
    out_cp(CHUNKS - 1, (CHUNKS - 1) % NBUF).wait()


@jax.jit
def kernel(img):
    mesh = plsc.VectorSubcoreMesh(core_axis_name="c", subcore_axis_name="s")
    sc_k = pl.kernel(
        _sc_body,
        out_type=jax.ShapeDtypeStruct((ROWS, COLS), jnp.float32),
        mesh=mesh,
        scratch_types=(
            [pltpu.VMEM((CH, COLS), jnp.float32) for _ in range(NBUF)]
            + [pltpu.SemaphoreType.DMA for _ in range(2 * NBUF)]
        ),
    )
    return sc_k(img)
